# Initial kernel scaffold; baseline (speedup 1.0000x reference)
import jax
import jax.numpy as jnp
from jax.experimental import pallas as pl


def kernel(node_emb, edges, edge_weight, W):
    raise NotImplementedError("write your pallas kernel here")


# speedup vs baseline: 3.6896x; 3.6896x over previous
import functools

import jax
import jax.numpy as jnp
from jax import lax
from jax.experimental import pallas as pl
from jax.experimental.pallas import tpu as pltpu
from jax.experimental.pallas import tpu_sc as plsc

N_NODES = 10000
N_PAD = 10240
D = 128
NC = 2
NS = 16
NT = NC * NS
CH = 128
LANES = 8


def _sc_aggregate(node_emb, dst, src, w, e_per_tile):
    n_chunks = e_per_tile // CH
    rows_per_tile = N_PAD // NS

    mesh = plsc.VectorSubcoreMesh(
        core_axis_name="c", subcore_axis_name="s", num_cores=NC, num_subcores=NS
    )

    @functools.partial(
        pl.kernel,
        out_type=jax.ShapeDtypeStruct((NC, N_PAD, D), jnp.float32),
        mesh=mesh,
        scratch_types=[
            pltpu.VMEM((CH,), jnp.int32),
            pltpu.VMEM((CH,), jnp.int32),
            pltpu.VMEM((CH,), jnp.float32),
            pltpu.VMEM((CH, D), jnp.float32),
            pltpu.VMEM_SHARED((N_PAD, D), jnp.float32),
            pltpu.SemaphoreType.DMA,
        ],
    )
    def k(emb_hbm, dst_hbm, src_hbm, w_hbm, out_hbm,
          src_v, dst_v, w_v, rows_v, acc, sem):
        c = lax.axis_index("c")
        s = lax.axis_index("s")
        t = s * NC + c

        zeros16 = jnp.zeros((16,), jnp.float32)

        def zero_row(i, carry):
            for j in range(LANES):
                rows_v[i, pl.ds(j * 16, 16)] = zeros16
            return carry

        lax.fori_loop(0, CH, zero_row, None)

        r0 = s * rows_per_tile
        for i in range(rows_per_tile // CH):
            pltpu.sync_copy(rows_v, acc.at[pl.ds(r0 + i * CH, CH)])
        plsc.subcore_barrier()

        def chunk_body(j, carry):
            base = t * e_per_tile + j * CH
            pltpu.sync_copy(src_hbm.at[pl.ds(base, CH)], src_v)
            pltpu.sync_copy(dst_hbm.at[pl.ds(base, CH)], dst_v)
            pltpu.sync_copy(w_hbm.at[pl.ds(base, CH)], w_v)
            pltpu.async_copy(emb_hbm.at[src_v], rows_v, sem).wait()

            def scale_group(g, carry2):
                w16 = w_v[pl.ds(g * 16, 16)]

                def scale_e(e16, carry3):
                    idx = jnp.full((16, 1), e16, jnp.int32)
                    we = lax.gather(
                        w16, idx,
                        lax.GatherDimensionNumbers(
                            offset_dims=(), collapsed_slice_dims=(0,),
                            start_index_map=(0,)),
                        (1,),
                        mode=lax.GatherScatterMode.PROMISE_IN_BOUNDS)
                    e = g * 16 + e16
                    for jj in range(LANES):
                        sl = pl.ds(jj * 16, 16)
                        rows_v[e, sl] = rows_v[e, sl] * we
                    return carry3

                lax.fori_loop(0, 16, scale_e, None)
                return carry2

            lax.fori_loop(0, CH // 16, scale_group, None)
            pltpu.sync_copy(rows_v, acc.at[dst_v], add=True)
            return carry

        lax.fori_loop(0, n_chunks, chunk_body, None)
        plsc.subcore_barrier()

        pltpu.sync_copy(acc.at[pl.ds(r0, rows_per_tile)],
                        out_hbm.at[c, pl.ds(r0, rows_per_tile)])

    return k(node_emb, dst, src, w)


def _tc_finish(partials, W):
    BLK = 1000

    def body(p_ref, w_ref, o_ref):
        x = p_ref[0] + p_ref[1]
        o_ref[...] = jnp.dot(x, w_ref[...], preferred_element_type=jnp.float32)

    return pl.pallas_call(
        body,
        grid=(N_NODES // BLK,),
        in_specs=[
            pl.BlockSpec((NC, BLK, D), lambda i: (0, i, 0)),
            pl.BlockSpec((D, D), lambda i: (0, 0)),
        ],
        out_specs=pl.BlockSpec((BLK, D), lambda i: (i, 0)),
        out_shape=jax.ShapeDtypeStruct((N_NODES, D), jnp.float32),
    )(partials, W)


def kernel(node_emb, edges, edge_weight, W):
    E = edges.shape[1]
    e_per_tile = -(-E // (NT * CH)) * CH
    E_pad = e_per_tile * NT
    pad = E_pad - E
    dst = jnp.concatenate([edges[0], jnp.zeros((pad,), jnp.int32)])
    src = jnp.concatenate([edges[1], jnp.zeros((pad,), jnp.int32)])
    w = jnp.concatenate([edge_weight, jnp.zeros((pad,), jnp.float32)])
    partials = _sc_aggregate(node_emb, dst, src, w, e_per_tile)
    return _tc_finish(partials, W)
